# baseline (speedup 1.0000x reference)
import functools

import jax
import jax.numpy as jnp
from jax.experimental import pallas as pl

F32 = jnp.float32


def _lrelu(x):
    return jnp.where(x >= 0, x, 0.2 * x)


def _conv_body(x_ref, w_ref, b_ref, o_ref, *, taps, TH, Wout, Ci, Co,
               pool, act):
    t = pl.program_id(1)
    h0 = t * TH
    acc = jnp.zeros((TH * Wout, Co), F32)
    for i, (dy, dx) in enumerate(taps):
        xs = x_ref[0, pl.ds(h0 + dy, TH), dx:dx + Wout, :]
        xs = xs.reshape(TH * Wout, Ci)
        acc = acc + jnp.dot(xs, w_ref[i], preferred_element_type=F32)
    y = acc + b_ref[0][None, :]
    y = y.reshape(TH, Wout, Co)
    if pool:
        y = y.reshape(TH // 2, 2, Wout, Co)
        y = jnp.max(y, axis=1)
        y = y.reshape(TH // 2, Wout // 2, 2, Co)
        y = jnp.max(y, axis=2)
    y = act(y)
    o_ref[0] = y


def _conv(x, w, b, *, pool, act, TH=28):
    kh, kw, Ci, Co = w.shape
    B, Hp, Wp = x.shape[0], x.shape[1], x.shape[2]
    Hout, Wout = Hp - kh + 1, Wp - kw + 1
    taps = [(dy, dx) for dy in range(kh) for dx in range(kw)]
    wr = w.reshape(kh * kw, Ci, Co)
    br = b.reshape(1, Co)
    THo, Wo = (TH // 2, Wout // 2) if pool else (TH, Wout)
    body = functools.partial(_conv_body, taps=taps, TH=TH, Wout=Wout,
                             Ci=Ci, Co=Co, pool=pool, act=act)
    return pl.pallas_call(
        body,
        grid=(B, Hout // TH),
        in_specs=[
            pl.BlockSpec((1, Hp, Wp, Ci), lambda bb, tt: (bb, 0, 0, 0)),
            pl.BlockSpec(wr.shape, lambda bb, tt: (0, 0, 0)),
            pl.BlockSpec(br.shape, lambda bb, tt: (0, 0)),
        ],
        out_specs=pl.BlockSpec((1, THo, Wo, Co), lambda bb, tt: (bb, tt, 0, 0)),
        out_shape=jax.ShapeDtypeStruct((B, Hout // TH * THo, Wo, Co), F32),
    )(x, wr, br)


def _enc1_body(x_ref, w_ref, b_ref, o_ref, *, TH, Wout, Co):
    t = pl.program_id(1)
    h0 = t * TH
    bias = b_ref[0][None, :]
    y = None
    for pi, py in enumerate((0, 1)):
        for pj, px in enumerate((0, 1)):
            acc = jnp.zeros((TH * Wout, Co), F32)
            for a in range(2):
                for c in range(2):
                    rs = h0 + a + py
                    cs = c + px
                    xs = x_ref[0, pl.ds(rs, TH), cs:cs + Wout, :]
                    xs = xs.reshape(TH * Wout, 12)
                    wi = w_ref[((pi * 2 + pj) * 2 + a) * 2 + c]
                    acc = acc + jnp.dot(xs, wi, preferred_element_type=F32)
            y = acc if y is None else jnp.maximum(y, acc)
    y = _lrelu(y + bias)
    o_ref[0] = y.reshape(TH, Wout, Co)


def _enc1(x, w, b, *, TH=28):
    B = x.shape[0]
    Co = w.shape[3]
    xg = x.reshape(B, 112, 2, 112, 2, 3)
    xg = jnp.transpose(xg, (0, 1, 3, 2, 4, 5)).reshape(B, 112, 112, 12)
    xg = jnp.pad(xg, ((0, 0), (1, 1), (1, 1), (0, 0)))
    z34 = jnp.zeros((3, Co), F32)
    mats = []
    for py in (0, 1):
        for px in (0, 1):
            rmap = {}
            for e in (-1, 0, 1):
                t_g = (py + e) // 2
                a = t_g + 1 if py == 0 else t_g
                rmap[(a, (py + e) % 2)] = e + 1
            cmap = {}
            for e in (-1, 0, 1):
                t_g = (px + e) // 2
                c = t_g + 1 if px == 0 else t_g
                cmap[(c, (px + e) % 2)] = e + 1
            for a in range(2):
                for c in range(2):
                    blocks = []
                    for pr in range(2):
                        for pc in range(2):
                            dy = rmap.get((a, pr))
                            dx = cmap.get((c, pc))
                            if dy is None or dx is None:
                                blocks.append(z34)
                            else:
                                blocks.append(w[dy, dx])
                    mats.append(jnp.concatenate(blocks, axis=0))
    wg = jnp.stack(mats, axis=0)
    body = functools.partial(_enc1_body, TH=TH, Wout=112, Co=Co)
    return pl.pallas_call(
        body,
        grid=(B, 112 // TH),
        in_specs=[
            pl.BlockSpec((1, 114, 114, 12), lambda bb, tt: (bb, 0, 0, 0)),
            pl.BlockSpec(wg.shape, lambda bb, tt: (0, 0, 0)),
            pl.BlockSpec((1, Co), lambda bb, tt: (0, 0)),
        ],
        out_specs=pl.BlockSpec((1, TH, 112, Co), lambda bb, tt: (bb, tt, 0, 0)),
        out_shape=jax.ShapeDtypeStruct((B, 112, 112, Co), F32),
    )(xg, wg, b.reshape(1, Co))


def _deconv_body(x_ref, wf_ref, b_ref, o_ref, *, TH, W, Ci, Co):
    t = pl.program_id(1)
    i0 = t * TH
    xs = x_ref[0, pl.ds(i0, TH + 2), :, :]
    xs = xs.reshape((TH + 2) * (W + 2), Ci)
    u = jnp.dot(xs, wf_ref[...], preferred_element_type=F32)
    u = u.reshape(TH + 2, W + 2, 16 * Co)
    bias = b_ref[0]

    def up(r, s):
        k = (4 * r + s) * Co
        return u[:, :, k:k + Co]

    ph = {}
    for py in range(2):
        for px in range(2):
            v = bias[None, None, :]
            for a in range(2):
                for bb in range(2):
                    v = v + up(py + 2 * a, px + 2 * bb)[
                        py + a:py + a + TH, px + bb:px + bb + W]
            ph[(py, px)] = _lrelu(v)
    r0 = jnp.concatenate(
        [ph[(0, 0)][:, :, None, :], ph[(0, 1)][:, :, None, :]], axis=2)
    r1 = jnp.concatenate(
        [ph[(1, 0)][:, :, None, :], ph[(1, 1)][:, :, None, :]], axis=2)
    y = jnp.concatenate([r0[:, None], r1[:, None]], axis=1)
    o_ref[0] = y.reshape(2 * TH, 2 * W, Co)


def _deconv(x, w, b, *, TH=14):
    B, H, W, Ci = x.shape
    Co = w.shape[3]
    xp = jnp.pad(x, ((0, 0), (1, 1), (1, 1), (0, 0)))
    wf = jnp.transpose(w, (2, 0, 1, 3)).reshape(Ci, 16 * Co)
    br = b.reshape(1, Co)
    body = functools.partial(_deconv_body, TH=TH, W=W, Ci=Ci, Co=Co)
    return pl.pallas_call(
        body,
        grid=(B, H // TH),
        in_specs=[
            pl.BlockSpec((1, H + 2, W + 2, Ci), lambda bb, tt: (bb, 0, 0, 0)),
            pl.BlockSpec(wf.shape, lambda bb, tt: (0, 0)),
            pl.BlockSpec(br.shape, lambda bb, tt: (0, 0)),
        ],
        out_specs=pl.BlockSpec((1, 2 * TH, 2 * W, Co),
                               lambda bb, tt: (bb, tt, 0, 0)),
        out_shape=jax.ShapeDtypeStruct((B, 2 * H, 2 * W, Co), F32),
    )(xp, wf, br)


def _vq_body(x_ref, w_ref, b_ref, cbt_ref, cb_ref, o_ref):
    zp = jax.nn.sigmoid(
        jnp.dot(x_ref[...], w_ref[...], preferred_element_type=F32)
        + b_ref[0][None, :])
    cbt = cbt_ref[...]
    cbsq = jnp.sum(cbt * cbt, axis=0, keepdims=True)
    d = cbsq - 2.0 * jnp.dot(zp, cbt, preferred_element_type=F32)
    dmin = jnp.min(d, axis=1, keepdims=True)
    iota = jax.lax.broadcasted_iota(jnp.int32, d.shape, 1)
    big = jnp.int32(d.shape[1])
    masked = jnp.where(d <= dmin, iota, big)
    idx = jnp.min(masked, axis=1, keepdims=True)
    oh = (iota == idx).astype(F32)
    o_ref[...] = jnp.dot(oh, cb_ref[...], preferred_element_type=F32)


def _vq(z, w5, b5, codebook):
    M = z.shape[0]
    C = codebook.shape[1]
    return pl.pallas_call(
        _vq_body,
        out_shape=jax.ShapeDtypeStruct((M, C), F32),
    )(z, w5, b5.reshape(1, C), codebook.T, codebook)


def _final_body(x_ref, w1_ref, b1_ref, w2_ref, b2_ref, o_ref, *, TH, W):
    acc = jnp.zeros((TH * W, 32), F32)
    for i, (dy, dx) in enumerate(
            [(dy, dx) for dy in range(3) for dx in range(3)]):
        xs = x_ref[0, dy:dy + TH, dx:dx + W, :]
        xs = xs.reshape(TH * W, 64)
        acc = acc + jnp.dot(xs, w1_ref[i], preferred_element_type=F32)
    y = _lrelu(acc + b1_ref[0][None, :])
    y = jax.nn.sigmoid(
        jnp.dot(y, w2_ref[...], preferred_element_type=F32)
        + b2_ref[0][None, :])
    o_ref[0] = y.reshape(TH, W, 3)


def _final(x, w1, b1, w2, b2, *, TH=28):
    B, Hp, Wp, _ = x.shape
    H, W = Hp - 2, Wp - 2
    nt = H // TH
    xt = jnp.stack([x[:, i * TH:i * TH + TH + 2] for i in range(nt)], axis=1)
    xt = xt.reshape(B * nt, TH + 2, Wp, 64)
    w1r = w1.reshape(9, 64, 32)
    w2r = w2.reshape(32, 3)
    body = functools.partial(_final_body, TH=TH, W=W)
    y = pl.pallas_call(
        body,
        grid=(B * nt,),
        in_specs=[
            pl.BlockSpec((1, TH + 2, Wp, 64), lambda g: (g, 0, 0, 0)),
            pl.BlockSpec(w1r.shape, lambda g: (0, 0, 0)),
            pl.BlockSpec((1, 32), lambda g: (0, 0)),
            pl.BlockSpec(w2r.shape, lambda g: (0, 0)),
            pl.BlockSpec((1, 3), lambda g: (0, 0)),
        ],
        out_specs=pl.BlockSpec((1, TH, W, 3), lambda g: (g, 0, 0, 0)),
        out_shape=jax.ShapeDtypeStruct((B * nt, TH, W, 3), F32),
    )(xt, w1r, b1.reshape(1, 32), w2r, b2.reshape(1, 3))
    return y.reshape(B, H, W, 3)


def kernel(input, enc_params, dec_deconv, dec_conv, codebook):
    x = jnp.transpose(input, (0, 2, 3, 1))
    B = x.shape[0]
    h = _enc1(x, enc_params[0][0], enc_params[0][1])
    for i in (1, 2, 3):
        hp = jnp.pad(h, ((0, 0), (1, 1), (1, 1), (0, 0)))
        h = _conv(hp, enc_params[i][0], enc_params[i][1], pool=True,
                  act=_lrelu)
    w5, b5 = enc_params[4]
    M = B * h.shape[1] * h.shape[2]
    q = _vq(h.reshape(M, h.shape[3]), w5.reshape(w5.shape[2], w5.shape[3]),
            b5, codebook)
    qz = q.reshape(B, h.shape[1], h.shape[2], codebook.shape[1])
    for (w, b) in dec_deconv:
        qz = _deconv(qz, w, b)
    qp = jnp.pad(qz, ((0, 0), (1, 1), (1, 1), (0, 0)))
    y = _final(qp, dec_conv[0][0], dec_conv[0][1], dec_conv[1][0],
               dec_conv[1][1])
    return jnp.transpose(y, (0, 3, 1, 2))


# speedup vs baseline: 1.1065x; 1.1065x over previous
import functools

import jax
import jax.numpy as jnp
from jax.experimental import pallas as pl

F32 = jnp.float32


def _lrelu(x):
    return jnp.where(x >= 0, x, 0.2 * x)


def _conv_body(x_ref, w_ref, b_ref, o_ref, *, taps, TH, Wout, Ci, Co,
               pool, act):
    t = pl.program_id(1)
    h0 = t * TH
    acc = jnp.zeros((TH * Wout, Co), F32)
    for i, (dy, dx) in enumerate(taps):
        xs = x_ref[0, pl.ds(h0 + dy, TH), dx:dx + Wout, :]
        xs = xs.reshape(TH * Wout, Ci)
        acc = acc + jnp.dot(xs, w_ref[i], preferred_element_type=F32)
    y = acc + b_ref[0][None, :]
    y = y.reshape(TH, Wout, Co)
    if pool:
        y = y.reshape(TH // 2, 2, Wout, Co)
        y = jnp.max(y, axis=1)
        y = y.reshape(TH // 2, Wout // 2, 2, Co)
        y = jnp.max(y, axis=2)
    y = act(y)
    o_ref[0] = y


def _conv(x, w, b, *, pool, act, TH=28):
    kh, kw, Ci, Co = w.shape
    B, Hp, Wp = x.shape[0], x.shape[1], x.shape[2]
    Hout, Wout = Hp - kh + 1, Wp - kw + 1
    taps = [(dy, dx) for dy in range(kh) for dx in range(kw)]
    wr = w.reshape(kh * kw, Ci, Co)
    br = b.reshape(1, Co)
    THo, Wo = (TH // 2, Wout // 2) if pool else (TH, Wout)
    body = functools.partial(_conv_body, taps=taps, TH=TH, Wout=Wout,
                             Ci=Ci, Co=Co, pool=pool, act=act)
    return pl.pallas_call(
        body,
        grid=(B, Hout // TH),
        in_specs=[
            pl.BlockSpec((1, Hp, Wp, Ci), lambda bb, tt: (bb, 0, 0, 0)),
            pl.BlockSpec(wr.shape, lambda bb, tt: (0, 0, 0)),
            pl.BlockSpec(br.shape, lambda bb, tt: (0, 0)),
        ],
        out_specs=pl.BlockSpec((1, THo, Wo, Co), lambda bb, tt: (bb, tt, 0, 0)),
        out_shape=jax.ShapeDtypeStruct((B, Hout // TH * THo, Wo, Co), F32),
    )(x, wr, br)


def _enc1_body(x_ref, w_ref, b_ref, o_ref, *, TH, Wout, Co):
    t = pl.program_id(1)
    h0 = t * TH
    bias = b_ref[0][None, :]
    y = None
    for pi, py in enumerate((0, 1)):
        for pj, px in enumerate((0, 1)):
            acc = jnp.zeros((TH * Wout, Co), F32)
            for a in range(2):
                for c in range(2):
                    rs = h0 + a + py
                    cs = c + px
                    xs = x_ref[0, pl.ds(rs, TH), cs:cs + Wout, :]
                    xs = xs.reshape(TH * Wout, 12)
                    wi = w_ref[((pi * 2 + pj) * 2 + a) * 2 + c]
                    acc = acc + jnp.dot(xs, wi, preferred_element_type=F32)
            y = acc if y is None else jnp.maximum(y, acc)
    y = _lrelu(y + bias)
    o_ref[0] = y.reshape(TH, Wout, Co)


def _grouped3x3_mats(w):
    Ci, Co = w.shape[2], w.shape[3]
    zb = jnp.zeros((Ci, Co), F32)
    mats = []
    for py in (0, 1):
        for px in (0, 1):
            rmap = {}
            for e in (-1, 0, 1):
                t_g = (py + e) // 2
                a = t_g + 1 if py == 0 else t_g
                rmap[(a, (py + e) % 2)] = e + 1
            cmap = {}
            for e in (-1, 0, 1):
                t_g = (px + e) // 2
                c = t_g + 1 if px == 0 else t_g
                cmap[(c, (px + e) % 2)] = e + 1
            for a in range(2):
                for c in range(2):
                    blocks = []
                    for pr in range(2):
                        for pc in range(2):
                            dy = rmap.get((a, pr))
                            dx = cmap.get((c, pc))
                            if dy is None or dx is None:
                                blocks.append(zb)
                            else:
                                blocks.append(w[dy, dx])
                    mats.append(jnp.concatenate(blocks, axis=0))
    return jnp.stack(mats, axis=0)


def _enc1(x, w, b, *, TH=28):
    B = x.shape[0]
    Co = w.shape[3]
    xg = x.reshape(B, 112, 2, 112, 2, 3)
    xg = jnp.transpose(xg, (0, 1, 3, 2, 4, 5)).reshape(B, 112, 112, 12)
    xg = jnp.pad(xg, ((0, 0), (1, 1), (1, 1), (0, 0)))
    wg = _grouped3x3_mats(w)
    body = functools.partial(_enc1_body, TH=TH, Wout=112, Co=Co)
    return pl.pallas_call(
        body,
        grid=(B, 112 // TH),
        in_specs=[
            pl.BlockSpec((1, 114, 114, 12), lambda bb, tt: (bb, 0, 0, 0)),
            pl.BlockSpec(wg.shape, lambda bb, tt: (0, 0, 0)),
            pl.BlockSpec((1, Co), lambda bb, tt: (0, 0)),
        ],
        out_specs=pl.BlockSpec((1, TH, 112, Co), lambda bb, tt: (bb, tt, 0, 0)),
        out_shape=jax.ShapeDtypeStruct((B, 112, 112, Co), F32),
    )(xg, wg, b.reshape(1, Co))


def _deconv_body(x_ref, wf_ref, b_ref, o_ref, *, TH, W, Ci, Co, grouped_out):
    t = pl.program_id(1)
    i0 = t * TH
    xs = x_ref[0, pl.ds(i0, TH + 2), :, :]
    xs = xs.reshape((TH + 2) * (W + 2), Ci)
    u = jnp.dot(xs, wf_ref[...], preferred_element_type=F32)
    u = u.reshape(TH + 2, W + 2, 16 * Co)
    bias = b_ref[0]

    def up(r, s):
        k = (4 * r + s) * Co
        return u[:, :, k:k + Co]

    ph = {}
    for py in range(2):
        for px in range(2):
            v = bias[None, None, :]
            for a in range(2):
                for bb in range(2):
                    v = v + up(py + 2 * a, px + 2 * bb)[
                        py + a:py + a + TH, px + bb:px + bb + W]
            ph[(py, px)] = _lrelu(v)
    if grouped_out:
        y = jnp.concatenate(
            [ph[(0, 0)], ph[(0, 1)], ph[(1, 0)], ph[(1, 1)]], axis=-1)
        o_ref[0] = y
    else:
        r0 = jnp.concatenate(
            [ph[(0, 0)][:, :, None, :], ph[(0, 1)][:, :, None, :]], axis=2)
        r1 = jnp.concatenate(
            [ph[(1, 0)][:, :, None, :], ph[(1, 1)][:, :, None, :]], axis=2)
        y = jnp.concatenate([r0[:, None], r1[:, None]], axis=1)
        o_ref[0] = y.reshape(2 * TH, 2 * W, Co)


def _deconv(x, w, b, *, TH=14, grouped_out=False):
    B, H, W, Ci = x.shape
    Co = w.shape[3]
    xp = jnp.pad(x, ((0, 0), (1, 1), (1, 1), (0, 0)))
    wf = jnp.transpose(w, (2, 0, 1, 3)).reshape(Ci, 16 * Co)
    br = b.reshape(1, Co)
    body = functools.partial(_deconv_body, TH=TH, W=W, Ci=Ci, Co=Co,
                             grouped_out=grouped_out)
    if grouped_out:
        ospec = pl.BlockSpec((1, TH, W, 4 * Co), lambda bb, tt: (bb, tt, 0, 0))
        oshape = jax.ShapeDtypeStruct((B, H, W, 4 * Co), F32)
    else:
        ospec = pl.BlockSpec((1, 2 * TH, 2 * W, Co),
                             lambda bb, tt: (bb, tt, 0, 0))
        oshape = jax.ShapeDtypeStruct((B, 2 * H, 2 * W, Co), F32)
    return pl.pallas_call(
        body,
        grid=(B, H // TH),
        in_specs=[
            pl.BlockSpec((1, H + 2, W + 2, Ci), lambda bb, tt: (bb, 0, 0, 0)),
            pl.BlockSpec(wf.shape, lambda bb, tt: (0, 0)),
            pl.BlockSpec(br.shape, lambda bb, tt: (0, 0)),
        ],
        out_specs=ospec,
        out_shape=oshape,
    )(xp, wf, br)


def _vq_body(x_ref, w_ref, b_ref, cbt_ref, cb_ref, o_ref):
    zp = jax.nn.sigmoid(
        jnp.dot(x_ref[...], w_ref[...], preferred_element_type=F32)
        + b_ref[0][None, :])
    cbt = cbt_ref[...]
    cbsq = jnp.sum(cbt * cbt, axis=0, keepdims=True)
    d = cbsq - 2.0 * jnp.dot(zp, cbt, preferred_element_type=F32)
    dmin = jnp.min(d, axis=1, keepdims=True)
    iota = jax.lax.broadcasted_iota(jnp.int32, d.shape, 1)
    big = jnp.int32(d.shape[1])
    masked = jnp.where(d <= dmin, iota, big)
    idx = jnp.min(masked, axis=1, keepdims=True)
    oh = (iota == idx).astype(F32)
    o_ref[...] = jnp.dot(oh, cb_ref[...], preferred_element_type=F32)


def _vq(z, w5, b5, codebook):
    M = z.shape[0]
    C = codebook.shape[1]
    return pl.pallas_call(
        _vq_body,
        out_shape=jax.ShapeDtypeStruct((M, C), F32),
    )(z, w5, b5.reshape(1, C), codebook.T, codebook)


def _final_body(x_ref, w9_ref, b1_ref, w2_ref, b2_ref, o_ref, *, TH, Wout):
    acc = jnp.zeros((TH * Wout, 128), F32)
    for rs in range(3):
        for cs in range(3):
            xs = x_ref[0, rs:rs + TH, cs:cs + Wout, :]
            xs = xs.reshape(TH * Wout, 256)
            acc = acc + jnp.dot(xs, w9_ref[rs * 3 + cs],
                                preferred_element_type=F32)
    y = _lrelu(acc + b1_ref[0][None, :])
    z = jax.nn.sigmoid(
        jnp.dot(y, w2_ref[...], preferred_element_type=F32)
        + b2_ref[0][None, :])
    o_ref[0] = z.reshape(TH, Wout, 12)


def _final(xg, w1, b1, w2, b2, *, TH=28):
    B = xg.shape[0]
    Wg = xg.shape[2]
    xp = jnp.pad(xg, ((0, 0), (1, 1), (1, 1), (0, 0)))
    nt = 112 // TH
    xt = jnp.stack([xp[:, i * TH:i * TH + TH + 2] for i in range(nt)], axis=1)
    xt = xt.reshape(B * nt, TH + 2, Wg + 2, 256)
    mats = _grouped3x3_mats(w1)
    z32 = jnp.zeros((256, 32), F32)
    w9 = []
    for rs in range(3):
        for cs in range(3):
            cols = []
            for py in (0, 1):
                for px in (0, 1):
                    a, c = rs - py, cs - px
                    if 0 <= a <= 1 and 0 <= c <= 1:
                        cols.append(mats[((py * 2 + px) * 2 + a) * 2 + c])
                    else:
                        cols.append(z32)
            w9.append(jnp.concatenate(cols, axis=1))
    w9 = jnp.stack(w9, axis=0)
    b1t = jnp.tile(b1, 4).reshape(1, 128)
    w2bd = jax.scipy.linalg.block_diag(*([w2.reshape(32, 3)] * 4))
    b2t = jnp.tile(b2, 4).reshape(1, 12)
    body = functools.partial(_final_body, TH=TH, Wout=Wg)
    y = pl.pallas_call(
        body,
        grid=(B * nt,),
        in_specs=[
            pl.BlockSpec((1, TH + 2, Wg + 2, 256), lambda g: (g, 0, 0, 0)),
            pl.BlockSpec(w9.shape, lambda g: (0, 0, 0)),
            pl.BlockSpec((1, 128), lambda g: (0, 0)),
            pl.BlockSpec(w2bd.shape, lambda g: (0, 0)),
            pl.BlockSpec((1, 12), lambda g: (0, 0)),
        ],
        out_specs=pl.BlockSpec((1, TH, Wg, 12), lambda g: (g, 0, 0, 0)),
        out_shape=jax.ShapeDtypeStruct((B * nt, TH, Wg, 12), F32),
    )(xt, w9, b1t, w2bd, b2t)
    return y.reshape(B, 112, 112, 12)


def kernel(input, enc_params, dec_deconv, dec_conv, codebook):
    x = jnp.transpose(input, (0, 2, 3, 1))
    B = x.shape[0]
    h = _enc1(x, enc_params[0][0], enc_params[0][1])
    for i in (1, 2, 3):
        hp = jnp.pad(h, ((0, 0), (1, 1), (1, 1), (0, 0)))
        h = _conv(hp, enc_params[i][0], enc_params[i][1], pool=True,
                  act=_lrelu)
    w5, b5 = enc_params[4]
    M = B * h.shape[1] * h.shape[2]
    q = _vq(h.reshape(M, h.shape[3]), w5.reshape(w5.shape[2], w5.shape[3]),
            b5, codebook)
    qz = q.reshape(B, h.shape[1], h.shape[2], codebook.shape[1])
    for (w, b) in dec_deconv[:3]:
        qz = _deconv(qz, w, b)
    qz = _deconv(qz, dec_deconv[3][0], dec_deconv[3][1], grouped_out=True)
    y = _final(qz, dec_conv[0][0], dec_conv[0][1], dec_conv[1][0],
               dec_conv[1][1])
    y = y.reshape(B, 112, 112, 2, 2, 3)
    return jnp.transpose(y, (0, 5, 1, 3, 2, 4)).reshape(B, 3, 224, 224)
